# Initial kernel scaffold; baseline (speedup 1.0000x reference)
import jax
import jax.numpy as jnp
from jax.experimental import pallas as pl


def kernel(point_cloud, detect_point, W1, b1, W2, b2, W1f, b1f, W2f, b2f):
    raise NotImplementedError("write your pallas kernel here")


# speedup vs baseline: 1.1785x; 1.1785x over previous
import jax
import jax.numpy as jnp
from jax.experimental import pallas as pl


def _fused_kernel(det_ref, detm2_ref, dT_ref, cT_ref, w1t_ref, b1_ref,
                  w2t_ref, b2_ref, w1tf_ref, b1f_ref, w2tf_ref, b2f_ref,
                  out_ref):
    N = det_ref.shape[1]
    M = cT_ref.shape[2]
    TN = 512
    k_close = N * 2 // 3

    c0 = cT_ref[0, 0:1, :]
    c1 = cT_ref[0, 1:2, :]
    c2 = cT_ref[0, 2:3, :]
    cn = (c0 * c0 + c1 * c1) + c2 * c2
    min_cols = []
    for t in range(N // TN):
        dm2 = detm2_ref[0, pl.ds(t * TN, TN), :]
        g = jax.lax.dot_general(dm2, cT_ref[0],
                                (((1,), (0,)), ((), ())),
                                preferred_element_type=jnp.float32)
        x0 = det_ref[0, pl.ds(t * TN, TN), 0:1]
        x1 = det_ref[0, pl.ds(t * TN, TN), 1:2]
        x2 = det_ref[0, pl.ds(t * TN, TN), 2:3]
        dn = (x0 * x0 + x1 * x1) + x2 * x2
        v = g + dn
        v = v + cn
        min_cols.append(jnp.min(v, axis=1, keepdims=True))
    mins_col = jnp.concatenate(min_cols, axis=0)
    mins_row = jnp.reshape(mins_col, (1, N))
    dis = jnp.sqrt(jnp.maximum(mins_row, 0.0))

    ib = jax.lax.bitcast_convert_type(dis, jnp.int32)
    ib = jnp.bitwise_and(ib, jnp.int32(0x7FFFFFFF))
    idxv = jax.lax.broadcasted_iota(jnp.int32, (1, N), 1)

    def vbody(i, lohi):
        lo, hi = lohi
        mid = lo + ((hi - lo) >> 1)
        cnt = jnp.sum((ib <= mid).astype(jnp.int32))
        pred = cnt >= k_close
        return (jnp.where(pred, lo, mid), jnp.where(pred, mid, hi))

    lo0 = jnp.int32(-1)
    hi0 = jnp.int32(0x7F800000)
    _, tau = jax.lax.fori_loop(0, 31, vbody, (lo0, hi0))

    c_less = jnp.sum((ib < tau).astype(jnp.int32))
    t_need = k_close - c_less
    eq = ib == tau

    def ibody(i, lohi):
        lo, hi = lohi
        mid = lo + ((hi - lo) >> 1)
        cnt = jnp.sum((eq & (idxv <= mid)).astype(jnp.int32))
        pred = cnt >= t_need
        return (jnp.where(pred, lo, mid), jnp.where(pred, mid, hi))

    _, i_star = jax.lax.fori_loop(0, 12, ibody, (jnp.int32(-1), jnp.int32(N - 1)))
    close = (ib < tau) | (eq & (idxv <= i_star))

    xr0 = dT_ref[0, 0:1, :]
    xr1 = dT_ref[0, 1:2, :]
    xr2 = dT_ref[0, 2:3, :]

    def mlp(w1t, b1, w2t, b2):
        h = w1t[:, 0:1] * xr0
        h = h + w1t[:, 1:2] * xr1
        h = h + w1t[:, 2:3] * xr2
        h = jnp.maximum(h + b1, 0.0)
        l = jax.lax.dot_general(w2t, h, (((1,), (0,)), ((), ())),
                                precision=jax.lax.Precision.HIGHEST,
                                preferred_element_type=jnp.float32)
        return l + b2

    lc = mlp(w1t_ref[...], b1_ref[...], w2t_ref[...], b2_ref[...])
    lf = mlp(w1tf_ref[...], b1f_ref[...], w2tf_ref[...], b2f_ref[...])
    out_ref[0] = jnp.where(close, lc, lf)


def kernel(point_cloud, detect_point, W1, b1, W2, b2, W1f, b1f, W2f, b2f):
    B, N, _ = detect_point.shape
    M = point_cloud.shape[1]
    H = W1.shape[1]
    dT = jnp.swapaxes(detect_point, 1, 2)
    cT = jnp.swapaxes(point_cloud, 1, 2)
    detm2 = detect_point * (-2.0)

    out = pl.pallas_call(
        _fused_kernel,
        grid=(B,),
        in_specs=[
            pl.BlockSpec((1, N, 3), lambda b: (b, 0, 0)),
            pl.BlockSpec((1, N, 3), lambda b: (b, 0, 0)),
            pl.BlockSpec((1, 3, N), lambda b: (b, 0, 0)),
            pl.BlockSpec((1, 3, M), lambda b: (b, 0, 0)),
            pl.BlockSpec((H, 3), lambda b: (0, 0)),
            pl.BlockSpec((H, 1), lambda b: (0, 0)),
            pl.BlockSpec((2, H), lambda b: (0, 0)),
            pl.BlockSpec((2, 1), lambda b: (0, 0)),
            pl.BlockSpec((H, 3), lambda b: (0, 0)),
            pl.BlockSpec((H, 1), lambda b: (0, 0)),
            pl.BlockSpec((2, H), lambda b: (0, 0)),
            pl.BlockSpec((2, 1), lambda b: (0, 0)),
        ],
        out_specs=pl.BlockSpec((1, 2, N), lambda b: (b, 0, 0)),
        out_shape=jax.ShapeDtypeStruct((B, 2, N), jnp.float32),
    )(detect_point, detm2, dT, cT,
      W1.T, b1.reshape(H, 1), W2.T, b2.reshape(2, 1),
      W1f.T, b1f.reshape(H, 1), W2f.T, b2f.reshape(2, 1))
    return jnp.swapaxes(out, 1, 2)


# speedup vs baseline: 3.2290x; 2.7398x over previous
import jax
import jax.numpy as jnp
from jax.experimental import pallas as pl

_TN = 512


def _dist_kernel(det_ref, detm2_ref, cT_ref, dis_ref):
    M = cT_ref.shape[2]
    c0 = cT_ref[0, 0:1, :]
    c1 = cT_ref[0, 1:2, :]
    c2 = cT_ref[0, 2:3, :]
    cn = (c0 * c0 + c1 * c1) + c2 * c2
    g = jax.lax.dot_general(detm2_ref[0], cT_ref[0],
                            (((1,), (0,)), ((), ())),
                            preferred_element_type=jnp.float32)
    x0 = det_ref[0, :, 0:1]
    x1 = det_ref[0, :, 1:2]
    x2 = det_ref[0, :, 2:3]
    dn = (x0 * x0 + x1 * x1) + x2 * x2
    v = g + dn
    v = v + cn
    rowmin = jnp.min(v, axis=1, keepdims=True)
    dis_ref[0] = jnp.sqrt(jnp.maximum(jnp.reshape(rowmin, (1, _TN)), 0.0))


def _select_mlp_kernel(dis_ref, dT_ref, w1t_ref, b1_ref, w2t_ref, b2_ref,
                       w1tf_ref, b1f_ref, w2tf_ref, b2f_ref, out_ref):
    N = dT_ref.shape[2]
    k_close = N * 2 // 3

    dis = dis_ref[0]
    ib = jax.lax.bitcast_convert_type(dis, jnp.int32)
    ib = jnp.bitwise_and(ib, jnp.int32(0x7FFFFFFF))
    idxv = jax.lax.broadcasted_iota(jnp.int32, (1, N), 1)
    jcol = jax.lax.broadcasted_iota(jnp.int32, (32, 1), 0) + 1

    base = jnp.full((1, 1), -1, dtype=jnp.int32)
    for stride in (1 << 26, 1 << 21, 1 << 16, 1 << 11, 1 << 6, 2, 1):
        thr = base + jcol * stride
        cnt = jnp.sum((ib <= thr).astype(jnp.int32), axis=1, keepdims=True)
        jstar = jnp.sum((cnt < k_close).astype(jnp.int32), axis=0,
                        keepdims=True)
        base = base + jstar * stride
    tau = base + 1

    c_less = jnp.sum((ib < tau).astype(jnp.int32), axis=1, keepdims=True)
    t_need = k_close - c_less
    eq = ib == tau

    base2 = jnp.full((1, 1), -1, dtype=jnp.int32)
    for stride in (128, 4, 1):
        thr = base2 + jcol * stride
        cnt = jnp.sum((eq & (idxv <= thr)).astype(jnp.int32), axis=1,
                      keepdims=True)
        jstar = jnp.sum((cnt < t_need).astype(jnp.int32), axis=0,
                        keepdims=True)
        base2 = base2 + jstar * stride
    i_star = base2 + 1
    close = (ib < tau) | (eq & (idxv <= i_star))

    xr0 = dT_ref[0, 0:1, :]
    xr1 = dT_ref[0, 1:2, :]
    xr2 = dT_ref[0, 2:3, :]

    def mlp(w1t, b1, w2t, b2):
        h = w1t[:, 0:1] * xr0
        h = h + w1t[:, 1:2] * xr1
        h = h + w1t[:, 2:3] * xr2
        h = jnp.maximum(h + b1, 0.0)
        l = jax.lax.dot_general(w2t, h, (((1,), (0,)), ((), ())),
                                precision=jax.lax.Precision.HIGHEST,
                                preferred_element_type=jnp.float32)
        return l + b2

    lc = mlp(w1t_ref[...], b1_ref[...], w2t_ref[...], b2_ref[...])
    lf = mlp(w1tf_ref[...], b1f_ref[...], w2tf_ref[...], b2f_ref[...])
    out_ref[0] = jnp.where(close, lc, lf)


def kernel(point_cloud, detect_point, W1, b1, W2, b2, W1f, b1f, W2f, b2f):
    B, N, _ = detect_point.shape
    M = point_cloud.shape[1]
    H = W1.shape[1]
    dT = jnp.swapaxes(detect_point, 1, 2)
    cT = jnp.swapaxes(point_cloud, 1, 2)
    detm2 = detect_point * (-2.0)

    dis = pl.pallas_call(
        _dist_kernel,
        grid=(B, N // _TN),
        in_specs=[
            pl.BlockSpec((1, _TN, 3), lambda b, t: (b, t, 0)),
            pl.BlockSpec((1, _TN, 3), lambda b, t: (b, t, 0)),
            pl.BlockSpec((1, 3, M), lambda b, t: (b, 0, 0)),
        ],
        out_specs=pl.BlockSpec((1, 1, _TN), lambda b, t: (b, 0, t)),
        out_shape=jax.ShapeDtypeStruct((B, 1, N), jnp.float32),
    )(detect_point, detm2, cT)

    out = pl.pallas_call(
        _select_mlp_kernel,
        grid=(B,),
        in_specs=[
            pl.BlockSpec((1, 1, N), lambda b: (b, 0, 0)),
            pl.BlockSpec((1, 3, N), lambda b: (b, 0, 0)),
            pl.BlockSpec((H, 3), lambda b: (0, 0)),
            pl.BlockSpec((H, 1), lambda b: (0, 0)),
            pl.BlockSpec((2, H), lambda b: (0, 0)),
            pl.BlockSpec((2, 1), lambda b: (0, 0)),
            pl.BlockSpec((H, 3), lambda b: (0, 0)),
            pl.BlockSpec((H, 1), lambda b: (0, 0)),
            pl.BlockSpec((2, H), lambda b: (0, 0)),
            pl.BlockSpec((2, 1), lambda b: (0, 0)),
        ],
        out_specs=pl.BlockSpec((1, 2, N), lambda b: (b, 0, 0)),
        out_shape=jax.ShapeDtypeStruct((B, 2, N), jnp.float32),
    )(dis, dT,
      W1.T, b1.reshape(H, 1), W2.T, b2.reshape(2, 1),
      W1f.T, b1f.reshape(H, 1), W2f.T, b2f.reshape(2, 1))
    return jnp.swapaxes(out, 1, 2)
